# baseline (speedup 1.0000x reference)
import jax
import jax.numpy as jnp
from jax import lax
from jax.experimental import pallas as pl
from jax.experimental.pallas import tpu as pltpu

NUM_VERBS = 25
NUM_HOIS = 234
BATCH = 16384
CHUNK = 1024
NCHUNKS = BATCH // CHUNK
NBUF = 3


def _gather_pipelined(idx_ref, in_hbm, out_hbm, in_bufs, out_bufs, in_sems, out_sems):
    verb_iota = lax.broadcasted_iota(jnp.int32, (NUM_VERBS, NUM_HOIS), 0)
    onehot = (idx_ref[0][None, :] == verb_iota).astype(jnp.float32)

    def in_copy(c):
        return pltpu.make_async_copy(
            in_hbm.at[pl.ds(c * CHUNK, CHUNK), :],
            in_bufs.at[c % NBUF],
            in_sems.at[c % NBUF],
        )

    def out_copy(c):
        return pltpu.make_async_copy(
            out_bufs.at[c % NBUF],
            out_hbm.at[pl.ds(c * CHUNK, CHUNK), :],
            out_sems.at[c % NBUF],
        )

    for c in range(min(NBUF, NCHUNKS)):
        in_copy(c).start()
    for c in range(NCHUNKS):
        in_copy(c).wait()
        if c >= NBUF:
            out_copy(c - NBUF).wait()
        out_bufs[c % NBUF] = jnp.dot(
            in_bufs[c % NBUF], onehot, preferred_element_type=jnp.float32
        )
        out_copy(c).start()
        if c + NBUF < NCHUNKS:
            in_copy(c + NBUF).start()
    for c in range(max(0, NCHUNKS - NBUF), NCHUNKS):
        out_copy(c).wait()


@jax.jit
def kernel(verb_scores, hoi_to_verb):
    return pl.pallas_call(
        _gather_pipelined,
        in_specs=[
            pl.BlockSpec(memory_space=pltpu.MemorySpace.VMEM),
            pl.BlockSpec(memory_space=pltpu.MemorySpace.HBM),
        ],
        out_specs=pl.BlockSpec(memory_space=pltpu.MemorySpace.HBM),
        out_shape=jax.ShapeDtypeStruct((BATCH, NUM_HOIS), jnp.float32),
        scratch_shapes=[
            pltpu.VMEM((NBUF, CHUNK, NUM_VERBS), jnp.float32),
            pltpu.VMEM((NBUF, CHUNK, NUM_HOIS), jnp.float32),
            pltpu.SemaphoreType.DMA((NBUF,)),
            pltpu.SemaphoreType.DMA((NBUF,)),
        ],
    )(hoi_to_verb.reshape(1, NUM_HOIS), verb_scores)


# speedup vs baseline: 1.0747x; 1.0747x over previous
import jax
import jax.numpy as jnp
from jax import lax
from jax.experimental import pallas as pl
from jax.experimental.pallas import tpu as pltpu

NUM_VERBS = 25
NUM_HOIS = 234
BATCH = 16384
BLOCK_B = 4096
NBLK = BATCH // BLOCK_B


def _gather_kernel(idx_ref, in_hbm, out_ref, in_bufs, in_sems):
    i = pl.program_id(0)
    verb_iota = lax.broadcasted_iota(jnp.int32, (NUM_VERBS, NUM_HOIS), 0)
    onehot = (idx_ref[0][None, :] == verb_iota).astype(jnp.float32)

    def in_copy(c):
        return pltpu.make_async_copy(
            in_hbm.at[pl.ds(c * BLOCK_B, BLOCK_B), :],
            in_bufs.at[lax.rem(c, 2)],
            in_sems.at[lax.rem(c, 2)],
        )

    @pl.when(i == 0)
    def _prologue():
        in_copy(0).start(priority=1)
        in_copy(1).start(priority=1)

    in_copy(i).wait()
    out_ref[...] = jnp.dot(
        in_bufs[lax.rem(i, 2)], onehot, preferred_element_type=jnp.float32
    )

    @pl.when(i + 2 < NBLK)
    def _prefetch():
        in_copy(i + 2).start(priority=1)


@jax.jit
def kernel(verb_scores, hoi_to_verb):
    return pl.pallas_call(
        _gather_kernel,
        grid=(NBLK,),
        in_specs=[
            pl.BlockSpec((1, NUM_HOIS), lambda i: (0, 0)),
            pl.BlockSpec(memory_space=pltpu.MemorySpace.HBM),
        ],
        out_specs=pl.BlockSpec((BLOCK_B, NUM_HOIS), lambda i: (i, 0)),
        out_shape=jax.ShapeDtypeStruct((BATCH, NUM_HOIS), jnp.float32),
        scratch_shapes=[
            pltpu.VMEM((2, BLOCK_B, NUM_VERBS), jnp.float32),
            pltpu.SemaphoreType.DMA((2,)),
        ],
        compiler_params=pltpu.CompilerParams(
            dimension_semantics=("arbitrary",),
        ),
    )(hoi_to_verb.reshape(1, NUM_HOIS), verb_scores)
